# baseline (device time: 121299 ns/iter reference)
import jax
import jax.numpy as jnp
from jax import lax
from jax.experimental import pallas as pl
from jax.experimental.pallas import tpu as pltpu

N_DEV = 4
NEG = -1e30


def kernel(Q, K, V, bt, lens):
    B, _, H, D = Q.shape
    NLOC, BS, _, _ = K.shape
    NB = bt.shape[1]
    CHUNK = 32
    NC = NLOC // CHUNK
    T = CHUNK * BS
    HD = H * D
    R = H * B
    scale = D ** -0.5

    def body(q_ref, k_ref, v_ref, btT_ref, lens_ref, out_ref,
             acc_ref, comm_ref, send_sems, recv_sems):
        c = pl.program_id(0)
        my = lax.axis_index("i")

        @pl.when(c == 0)
        def _init():
            z = jnp.zeros((R, D), jnp.float32)
            acc_ref[:, 0:D] = z
            acc_ref[:, D:2 * D] = z + NEG
            acc_ref[:, 2 * D:3 * D] = z

        base = my * NLOC + c * CHUNK
        pid = base + lax.broadcasted_iota(jnp.int32, (1, CHUNK), 1)
        ksub = lax.broadcasted_iota(jnp.int32, (NB, 1), 0)
        rows = []
        for i in range(B):
            btcol = btT_ref[:, i:i + 1]
            li = lens_ref[i:i + 1, 0:1]
            eq = (btcol == pid) & (ksub < li)
            rows.append(jnp.sum(eq.astype(jnp.float32), axis=0, keepdims=True))
        cnt_page = jnp.concatenate(rows, axis=0)
        expand = (
            lax.broadcasted_iota(jnp.int32, (CHUNK, T), 0)
            == lax.broadcasted_iota(jnp.int32, (CHUNK, T), 1) // BS
        ).astype(jnp.float32)
        cnt_tok = lax.dot_general(
            cnt_page, expand, (((1,), (0,)), ((), ())),
            preferred_element_type=jnp.float32,
        )
        cnt_all = jnp.concatenate([cnt_tok] * H, axis=0)
        pos_all = cnt_all > 0.0

        qv = q_ref[:, :]
        lane_h = lax.broadcasted_iota(jnp.int32, (1, HD), 1) // D
        q_all = jnp.concatenate(
            [jnp.where(lane_h == h, qv, 0.0) for h in range(H)], axis=0
        )

        kc = k_ref[:].reshape(T, HD)
        s = lax.dot_general(
            q_all, kc, (((1,), (1,)), ((), ())),
            preferred_element_type=jnp.float32,
        ) * scale
        s = jnp.where(pos_all, s, NEG)
        m_c = jnp.max(s, axis=1, keepdims=True)
        m_old = acc_ref[:, D:2 * D]
        m_new = jnp.maximum(m_old, m_c)
        alpha = jnp.exp(m_old - m_new)
        p = cnt_all * jnp.exp(s - m_new[:, 0:1])
        l_old = acc_ref[:, 2 * D:3 * D]
        l_new = l_old * alpha + jnp.sum(p, axis=1, keepdims=True)
        vc = v_ref[:].reshape(T, HD)
        o_full = lax.dot_general(
            p, vc, (((1,), (0,)), ((), ())),
            preferred_element_type=jnp.float32,
        )
        o_old = acc_ref[:, 0:D]
        o_scaled = o_old * alpha
        for h in range(H):
            acc_ref[h * B:(h + 1) * B, 0:D] = (
                o_scaled[h * B:(h + 1) * B, :]
                + o_full[h * B:(h + 1) * B, h * D:(h + 1) * D]
            )
        acc_ref[:, D:2 * D] = m_new
        acc_ref[:, 2 * D:3 * D] = l_new

        @pl.when(c == NC - 1)
        def _finish():
            for h in range(H):
                comm_ref[0, :, h, :] = acc_ref[h * B:(h + 1) * B, :]

            barrier = pltpu.get_barrier_semaphore()
            for t in range(1, N_DEV):
                pl.semaphore_signal(
                    barrier, inc=1,
                    device_id=((my + t) % N_DEV,),
                    device_id_type=pl.DeviceIdType.MESH,
                )
            pl.semaphore_wait(barrier, N_DEV - 1)

            rdmas = []
            for t in range(1, N_DEV):
                r = pltpu.make_async_remote_copy(
                    src_ref=comm_ref.at[0],
                    dst_ref=comm_ref.at[t],
                    send_sem=send_sems.at[t],
                    recv_sem=recv_sems.at[t],
                    device_id=((my + t) % N_DEV,),
                    device_id_type=pl.DeviceIdType.MESH,
                )
                r.start()
                rdmas.append(r)
            for r in rdmas:
                r.wait()

            ms = [comm_ref[t, :, :, D:2 * D] for t in range(N_DEV)]
            mg = jnp.maximum(jnp.maximum(ms[0], ms[1]),
                             jnp.maximum(ms[2], ms[3]))
            num = jnp.zeros((B, H, D), jnp.float32)
            den = jnp.zeros((B, H, D), jnp.float32)
            for t in range(N_DEV):
                w = jnp.exp(ms[t] - mg)
                num = num + comm_ref[t, :, :, 0:D] * w
                den = den + comm_ref[t, :, :, 2 * D:3 * D] * w
            out_ref[:, 0, :, :] = num / den

    btT = bt.T
    lens2 = lens.reshape(B, 1)
    Qf = Q.reshape(B, HD)
    Kf = K.reshape(NLOC, BS, HD)
    Vf = V.reshape(NLOC, BS, HD)

    return pl.pallas_call(
        body,
        grid=(NC,),
        in_specs=[
            pl.BlockSpec((B, HD), lambda c: (0, 0)),
            pl.BlockSpec((CHUNK, BS, HD), lambda c: (c, 0, 0)),
            pl.BlockSpec((CHUNK, BS, HD), lambda c: (c, 0, 0)),
            pl.BlockSpec((NB, B), lambda c: (0, 0)),
            pl.BlockSpec((B, 1), lambda c: (0, 0)),
        ],
        out_specs=pl.BlockSpec((B, 1, H, D), lambda c: (0, 0, 0, 0)),
        out_shape=jax.ShapeDtypeStruct((B, 1, H, D), jnp.float32),
        scratch_shapes=[
            pltpu.VMEM((R, 3 * D), jnp.float32),
            pltpu.VMEM((N_DEV, B, H, 3 * D), jnp.float32),
            pltpu.SemaphoreType.DMA((N_DEV,)),
            pltpu.SemaphoreType.DMA((N_DEV,)),
        ],
        compiler_params=pltpu.CompilerParams(
            collective_id=0,
            dimension_semantics=("arbitrary",),
        ),
    )(Qf, Kf, Vf, btT, lens2)


# device time: 107653 ns/iter; 1.1268x vs baseline; 1.1268x over previous
import jax
import jax.numpy as jnp
from jax import lax
from jax.experimental import pallas as pl
from jax.experimental.pallas import tpu as pltpu

N_DEV = 4


def kernel(Q, K, V, bt, lens):
    B, _, H, D = Q.shape
    NLOC, BS, _, _ = K.shape
    NB = bt.shape[1]
    CHUNK = 64
    NC = NLOC // CHUNK
    T = CHUNK * BS
    HD = H * D
    R = H * B
    scale = D ** -0.5

    def body(q_ref, k_ref, v_ref, btT_ref, lens_ref, out_ref,
             acc_ref, comm_ref, send_sems, recv_sems):
        c = pl.program_id(0)
        my = lax.axis_index("i")

        @pl.when(c == 0)
        def _init():
            acc_ref[:, :] = jnp.zeros((R, 2 * D), jnp.float32)

        base = my * NLOC + c * CHUNK
        pid = base + lax.broadcasted_iota(jnp.int32, (1, CHUNK), 1)
        ksub = lax.broadcasted_iota(jnp.int32, (NB, 1), 0)
        rows = []
        for i in range(B):
            btcol = btT_ref[:, i:i + 1]
            li = lens_ref[i:i + 1, 0:1]
            eq = (btcol == pid) & (ksub < li)
            rows.append(jnp.sum(eq.astype(jnp.float32), axis=0, keepdims=True))
        cnt_page = jnp.concatenate(rows, axis=0)
        expand = (
            lax.broadcasted_iota(jnp.int32, (CHUNK, T), 0)
            == lax.broadcasted_iota(jnp.int32, (CHUNK, T), 1) // BS
        ).astype(jnp.float32)
        cnt_tok = lax.dot_general(
            cnt_page, expand, (((1,), (0,)), ((), ())),
            preferred_element_type=jnp.float32,
        )
        cnt_all = jnp.concatenate([cnt_tok] * H, axis=0)

        qv = q_ref[:, :]
        lane_h = lax.broadcasted_iota(jnp.int32, (1, HD), 1) // D
        q_all = jnp.concatenate(
            [jnp.where(lane_h == h, qv, 0.0) for h in range(H)], axis=0
        )

        kc = k_ref[:].reshape(T, HD)
        s = lax.dot_general(
            q_all, kc, (((1,), (1,)), ((), ())),
            preferred_element_type=jnp.float32,
        ) * scale
        p = cnt_all * jnp.exp(s)
        vc = v_ref[:].reshape(T, HD)
        o_full = lax.dot_general(
            p, vc, (((1,), (0,)), ((), ())),
            preferred_element_type=jnp.float32,
        )
        acc_ref[:, D:2 * D] += jnp.sum(p, axis=1, keepdims=True)
        for h in range(H):
            acc_ref[h * B:(h + 1) * B, 0:D] += (
                o_full[h * B:(h + 1) * B, h * D:(h + 1) * D]
            )

        @pl.when(c == NC - 1)
        def _finish():
            for h in range(H):
                comm_ref[0, :, h, :] = acc_ref[h * B:(h + 1) * B, :]

            barrier = pltpu.get_barrier_semaphore()
            for t in range(1, N_DEV):
                pl.semaphore_signal(
                    barrier, inc=1,
                    device_id=((my + t) % N_DEV,),
                    device_id_type=pl.DeviceIdType.MESH,
                )
            pl.semaphore_wait(barrier, N_DEV - 1)

            rdmas = []
            for t in range(1, N_DEV):
                r = pltpu.make_async_remote_copy(
                    src_ref=comm_ref.at[0],
                    dst_ref=comm_ref.at[t],
                    send_sem=send_sems.at[t],
                    recv_sem=recv_sems.at[t],
                    device_id=((my + t) % N_DEV,),
                    device_id_type=pl.DeviceIdType.MESH,
                )
                r.start()
                rdmas.append(r)
            for r in rdmas:
                r.wait()

            num = jnp.zeros((B, H, D), jnp.float32)
            den = jnp.zeros((B, H, D), jnp.float32)
            for t in range(N_DEV):
                num = num + comm_ref[t, :, :, 0:D]
                den = den + comm_ref[t, :, :, D:2 * D]
            out_ref[:, 0, :, :] = num / den

    btT = bt.T
    lens2 = lens.reshape(B, 1)
    Qf = Q.reshape(B, HD)
    Kf = K.reshape(NLOC, BS, HD)
    Vf = V.reshape(NLOC, BS, HD)

    return pl.pallas_call(
        body,
        grid=(NC,),
        in_specs=[
            pl.BlockSpec((B, HD), lambda c: (0, 0)),
            pl.BlockSpec((CHUNK, BS, HD), lambda c: (c, 0, 0)),
            pl.BlockSpec((CHUNK, BS, HD), lambda c: (c, 0, 0)),
            pl.BlockSpec((NB, B), lambda c: (0, 0)),
            pl.BlockSpec((B, 1), lambda c: (0, 0)),
        ],
        out_specs=pl.BlockSpec((B, 1, H, D), lambda c: (0, 0, 0, 0)),
        out_shape=jax.ShapeDtypeStruct((B, 1, H, D), jnp.float32),
        scratch_shapes=[
            pltpu.VMEM((R, 2 * D), jnp.float32),
            pltpu.VMEM((N_DEV, B, H, 2 * D), jnp.float32),
            pltpu.SemaphoreType.DMA((N_DEV,)),
            pltpu.SemaphoreType.DMA((N_DEV,)),
        ],
        compiler_params=pltpu.CompilerParams(
            collective_id=0,
            dimension_semantics=("arbitrary",),
        ),
    )(Qf, Kf, Vf, btT, lens2)


# device time: 107340 ns/iter; 1.1300x vs baseline; 1.0029x over previous
import jax
import jax.numpy as jnp
from jax import lax
from jax.experimental import pallas as pl
from jax.experimental.pallas import tpu as pltpu

N_DEV = 4


def kernel(Q, K, V, bt, lens):
    B, _, H, D = Q.shape
    NLOC, BS, _, _ = K.shape
    NB = bt.shape[1]
    CHUNK = 64
    NC = NLOC // CHUNK
    T = CHUNK * BS
    HD = H * D
    R = H * B
    scale = D ** -0.5

    def body(q_ref, k_ref, v_ref, btT_ref, lens_ref, out_ref,
             acc_ref, comm_ref, send_sems, recv_sems):
        c = pl.program_id(0)
        my = lax.axis_index("i")

        @pl.when(c == 0)
        def _init():
            acc_ref[:, :] = jnp.zeros((R, 2 * D), jnp.float32)

        base = my * NLOC + c * CHUNK
        pid = base + lax.broadcasted_iota(jnp.int32, (1, CHUNK), 1)
        ksub = lax.broadcasted_iota(jnp.int32, (NB, 1), 0)
        rows = []
        for i in range(B):
            btcol = btT_ref[:, i:i + 1]
            li = lens_ref[i:i + 1, 0:1]
            eq = (btcol == pid) & (ksub < li)
            rows.append(jnp.sum(eq.astype(jnp.float32), axis=0, keepdims=True))
        cnt_page = jnp.concatenate(rows, axis=0)
        expand = (
            lax.broadcasted_iota(jnp.int32, (CHUNK, T), 0)
            == lax.broadcasted_iota(jnp.int32, (CHUNK, T), 1) // BS
        ).astype(jnp.float32)
        cnt_tok = lax.dot_general(
            cnt_page, expand, (((1,), (0,)), ((), ())),
            preferred_element_type=jnp.float32,
        )
        cnt_all = jnp.concatenate([cnt_tok] * H, axis=0)

        qv = q_ref[:, :]
        lane_h = lax.broadcasted_iota(jnp.int32, (1, HD), 1) // D
        q_all = jnp.concatenate(
            [jnp.where(lane_h == h, qv, 0.0) for h in range(H)], axis=0
        )

        kc = k_ref[:].reshape(T, HD)
        s = lax.dot_general(
            q_all, kc, (((1,), (1,)), ((), ())),
            preferred_element_type=jnp.float32,
        ) * scale
        p = cnt_all * jnp.exp(s)
        vc = v_ref[:].reshape(T, HD)
        o_full = lax.dot_general(
            p, vc, (((1,), (0,)), ((), ())),
            preferred_element_type=jnp.float32,
        )
        acc_ref[:, D:2 * D] += jnp.sum(p, axis=1, keepdims=True)
        for h in range(H):
            acc_ref[h * B:(h + 1) * B, 0:D] += (
                o_full[h * B:(h + 1) * B, h * D:(h + 1) * D]
            )

        @pl.when(c == NC - 2)
        def _barrier():
            barrier = pltpu.get_barrier_semaphore()
            for t in range(1, N_DEV):
                pl.semaphore_signal(
                    barrier, inc=1,
                    device_id=((my + t) % N_DEV,),
                    device_id_type=pl.DeviceIdType.MESH,
                )
            pl.semaphore_wait(barrier, N_DEV - 1)

        @pl.when(c == NC - 1)
        def _finish():
            for h in range(H):
                comm_ref[0, :, h, :] = acc_ref[h * B:(h + 1) * B, :]

            rdmas = []
            for t in range(1, N_DEV):
                r = pltpu.make_async_remote_copy(
                    src_ref=comm_ref.at[0],
                    dst_ref=comm_ref.at[t],
                    send_sem=send_sems.at[t],
                    recv_sem=recv_sems.at[t],
                    device_id=((my + t) % N_DEV,),
                    device_id_type=pl.DeviceIdType.MESH,
                )
                r.start()
                rdmas.append(r)
            for r in rdmas:
                r.wait()

            num = jnp.zeros((B, H, D), jnp.float32)
            den = jnp.zeros((B, H, D), jnp.float32)
            for t in range(N_DEV):
                num = num + comm_ref[t, :, :, 0:D]
                den = den + comm_ref[t, :, :, D:2 * D]
            out_ref[:, 0, :, :] = num / den

    btT = bt.T
    lens2 = lens.reshape(B, 1)
    Qf = Q.reshape(B, HD)
    Kf = K.reshape(NLOC, BS, HD)
    Vf = V.reshape(NLOC, BS, HD)

    return pl.pallas_call(
        body,
        grid=(NC,),
        in_specs=[
            pl.BlockSpec((B, HD), lambda c: (0, 0)),
            pl.BlockSpec((CHUNK, BS, HD), lambda c: (c, 0, 0)),
            pl.BlockSpec((CHUNK, BS, HD), lambda c: (c, 0, 0)),
            pl.BlockSpec((NB, B), lambda c: (0, 0)),
            pl.BlockSpec((B, 1), lambda c: (0, 0)),
        ],
        out_specs=pl.BlockSpec((B, 1, H, D), lambda c: (0, 0, 0, 0)),
        out_shape=jax.ShapeDtypeStruct((B, 1, H, D), jnp.float32),
        scratch_shapes=[
            pltpu.VMEM((R, 2 * D), jnp.float32),
            pltpu.VMEM((N_DEV, B, H, 2 * D), jnp.float32),
            pltpu.SemaphoreType.DMA((N_DEV,)),
            pltpu.SemaphoreType.DMA((N_DEV,)),
        ],
        compiler_params=pltpu.CompilerParams(
            collective_id=0,
            dimension_semantics=("arbitrary",),
        ),
    )(Qf, Kf, Vf, btT, lens2)


# device time: 106817 ns/iter; 1.1356x vs baseline; 1.0049x over previous
import jax
import jax.numpy as jnp
from jax import lax
from jax.experimental import pallas as pl
from jax.experimental.pallas import tpu as pltpu

N_DEV = 4


def kernel(Q, K, V, bt, lens):
    B, _, H, D = Q.shape
    NLOC, BS, _, _ = K.shape
    NB = bt.shape[1]
    G = 64
    NG = NLOC // G
    T = G * BS
    HD = H * D
    R = H * B
    scale = D ** -0.5

    my = lax.axis_index("i")
    base = my * NLOC
    pids = base + jnp.arange(NLOC, dtype=jnp.int32)
    valid = jnp.arange(NB, dtype=jnp.int32)[None, :] < lens[:, None]
    cnt = jnp.sum(
        (bt[:, :, None] == pids[None, None, :]) & valid[:, :, None],
        axis=1,
    ).astype(jnp.float32)
    ref = jnp.sum(cnt, axis=0) > 0.0
    order = jnp.argsort(~ref, stable=True).astype(jnp.int32)
    n_ref = jnp.sum(ref.astype(jnp.int32))
    ngroups = (n_ref + G - 1) // G
    cnt_tok = jnp.repeat(jnp.take(cnt, order, axis=1), BS, axis=1)
    cnt_grp = jnp.transpose(cnt_tok.reshape(B, NG, T), (1, 0, 2))
    ng_arr = jnp.reshape(ngroups, (1,)).astype(jnp.int32)

    def body(q_ref, k_any, v_any, order_ref, ng_ref, cnt_ref, out_ref,
             kbuf, vbuf, acc_ref, comm_ref, copy_sems, send_sems, recv_sems):
        my = lax.axis_index("i")
        ng = ng_ref[0]

        acc_ref[:, :] = jnp.zeros((R, 2 * D), jnp.float32)

        qv = q_ref[:, :]
        lane_h = lax.broadcasted_iota(jnp.int32, (1, HD), 1) // D
        q_all = jnp.concatenate(
            [jnp.where(lane_h == h, qv, 0.0) for h in range(H)], axis=0
        )

        def issue(g, buf):
            for j in range(G):
                pg = order_ref[g * G + j]
                pltpu.make_async_copy(
                    k_any.at[pg], kbuf.at[buf, j], copy_sems.at[buf]
                ).start()
                pltpu.make_async_copy(
                    v_any.at[pg], vbuf.at[buf, j], copy_sems.at[buf]
                ).start()

        def wait_group(buf):
            for j in range(G):
                pltpu.make_async_copy(
                    k_any.at[0], kbuf.at[buf, j], copy_sems.at[buf]
                ).wait()
                pltpu.make_async_copy(
                    v_any.at[0], vbuf.at[buf, j], copy_sems.at[buf]
                ).wait()

        @pl.when(ng > 0)
        def _prologue():
            issue(0, 0)

        def loop_body(g, _):
            buf = lax.rem(g, 2)
            nxt = lax.rem(g + 1, 2)

            @pl.when(g + 1 < ng)
            def _issue_next():
                issue(g + 1, nxt)

            wait_group(buf)

            kc = kbuf[buf].reshape(T, HD)
            s = lax.dot_general(
                q_all, kc, (((1,), (1,)), ((), ())),
                preferred_element_type=jnp.float32,
            ) * scale
            cnt_g = cnt_ref[g]
            cnt_all = jnp.concatenate([cnt_g] * H, axis=0)
            p = cnt_all * jnp.exp(s)
            vc = vbuf[buf].reshape(T, HD)
            o_full = lax.dot_general(
                p, vc, (((1,), (0,)), ((), ())),
                preferred_element_type=jnp.float32,
            )
            acc_ref[:, D:2 * D] += jnp.sum(p, axis=1, keepdims=True)
            for h in range(H):
                acc_ref[h * B:(h + 1) * B, 0:D] += (
                    o_full[h * B:(h + 1) * B, h * D:(h + 1) * D]
                )
            return 0

        lax.fori_loop(0, ng, loop_body, 0)

        for h in range(H):
            comm_ref[0, :, h, :] = acc_ref[h * B:(h + 1) * B, :]

        barrier = pltpu.get_barrier_semaphore()
        for t in range(1, N_DEV):
            pl.semaphore_signal(
                barrier, inc=1,
                device_id=((my + t) % N_DEV,),
                device_id_type=pl.DeviceIdType.MESH,
            )
        pl.semaphore_wait(barrier, N_DEV - 1)

        rdmas = []
        for t in range(1, N_DEV):
            r = pltpu.make_async_remote_copy(
                src_ref=comm_ref.at[0],
                dst_ref=comm_ref.at[t],
                send_sem=send_sems.at[t],
                recv_sem=recv_sems.at[t],
                device_id=((my + t) % N_DEV,),
                device_id_type=pl.DeviceIdType.MESH,
            )
            r.start()
            rdmas.append(r)
        for r in rdmas:
            r.wait()

        num = jnp.zeros((B, H, D), jnp.float32)
        den = jnp.zeros((B, H, D), jnp.float32)
        for t in range(N_DEV):
            num = num + comm_ref[t, :, :, 0:D]
            den = den + comm_ref[t, :, :, D:2 * D]
        out_ref[:, 0, :, :] = num / den

    Qf = Q.reshape(B, HD)
    Kf = K.reshape(NLOC, BS, HD)
    Vf = V.reshape(NLOC, BS, HD)

    return pl.pallas_call(
        body,
        in_specs=[
            pl.BlockSpec(memory_space=pltpu.MemorySpace.VMEM),
            pl.BlockSpec(memory_space=pl.ANY),
            pl.BlockSpec(memory_space=pl.ANY),
            pl.BlockSpec(memory_space=pltpu.MemorySpace.SMEM),
            pl.BlockSpec(memory_space=pltpu.MemorySpace.SMEM),
            pl.BlockSpec(memory_space=pltpu.MemorySpace.VMEM),
        ],
        out_specs=pl.BlockSpec(memory_space=pltpu.MemorySpace.VMEM),
        out_shape=jax.ShapeDtypeStruct((B, 1, H, D), jnp.float32),
        scratch_shapes=[
            pltpu.VMEM((2, G, BS, HD), jnp.float32),
            pltpu.VMEM((2, G, BS, HD), jnp.float32),
            pltpu.VMEM((R, 2 * D), jnp.float32),
            pltpu.VMEM((N_DEV, B, H, 2 * D), jnp.float32),
            pltpu.SemaphoreType.DMA((2,)),
            pltpu.SemaphoreType.DMA((N_DEV,)),
            pltpu.SemaphoreType.DMA((N_DEV,)),
        ],
        compiler_params=pltpu.CompilerParams(collective_id=0),
    )(Qf, Kf, Vf, order, ng_arr, cnt_grp)


# device time: 101581 ns/iter; 1.1941x vs baseline; 1.0515x over previous
import jax
import jax.numpy as jnp
from jax import lax
from jax.experimental import pallas as pl
from jax.experimental.pallas import tpu as pltpu

N_DEV = 4


def kernel(Q, K, V, bt, lens):
    B, _, H, D = Q.shape
    NLOC, BS, _, _ = K.shape
    NB = bt.shape[1]
    G = 64
    NG = NLOC // G
    T = G * BS
    HD = H * D
    R = H * B
    scale = D ** -0.5
    f32 = jnp.float32

    def body(q_ref, k_any, v_any, btT_ref, lens_ref, out_ref,
             kbuf, vbuf, acc_ref, comm_ref, cnt_grp, pg_vmem, pg_smem,
             stage_sem, copy_sems, send_sems, recv_sems):
        my = lax.axis_index("i")

        acc_ref[:, :] = jnp.zeros((R, 2 * D), f32)

        base = my * NLOC
        pid_col = base + lax.broadcasted_iota(jnp.int32, (NLOC, 1), 0)
        klane = lax.broadcasted_iota(jnp.int32, (1, NB), 1)
        cols = []
        for i in range(B):
            btrow = btT_ref[i:i + 1, :]
            li = lens_ref[i:i + 1, 0:1]
            eq = (btrow == pid_col) & (klane < li)
            cols.append(jnp.sum(eq.astype(f32), axis=1, keepdims=True))
        cntT = jnp.concatenate(cols, axis=1)
        ref_col = (jnp.sum(cntT, axis=1, keepdims=True) > 0.0).astype(f32)

        lt = (
            lax.broadcasted_iota(jnp.int32, (NLOC, NLOC), 1)
            < lax.broadcasted_iota(jnp.int32, (NLOC, NLOC), 0)
        ).astype(f32)
        rank_col = lax.dot_general(
            lt, ref_col, (((1,), (0,)), ((), ())),
            preferred_element_type=f32,
        )
        slot_row = lax.broadcasted_iota(jnp.int32, (1, NLOC), 1)
        sel = jnp.where(
            (rank_col == slot_row.astype(f32)) & (ref_col > 0.0), 1.0, 0.0
        )
        p_lane = lax.broadcasted_iota(jnp.int32, (1, NLOC), 1)
        p_hi = (p_lane // 256).astype(f32)
        p_lo = (p_lane % 256).astype(f32)
        pages = (
            256.0 * lax.dot_general(p_hi, sel, (((1,), (0,)), ((), ())),
                                    preferred_element_type=f32)
            + lax.dot_general(p_lo, sel, (((1,), (0,)), ((), ())),
                              preferred_element_type=f32)
        )
        n_ref = jnp.sum(ref_col)
        ng_f = jnp.floor((n_ref + (G - 1)) / G)
        pg_vmem[0:1, :] = pages.astype(jnp.int32)
        pg_vmem[1:2, :] = jnp.broadcast_to(
            ng_f.astype(jnp.int32)[None, None], (1, NLOC)
        )
        copy = pltpu.make_async_copy(pg_vmem, pg_smem, stage_sem)
        copy.start()

        cnt_sorted = lax.dot_general(
            cntT, sel, (((0,), (0,)), ((), ())),
            preferred_element_type=f32,
        )
        expand = (
            lax.broadcasted_iota(jnp.int32, (G, T), 0)
            == lax.broadcasted_iota(jnp.int32, (G, T), 1) // BS
        ).astype(f32)
        for g2 in range(NG):
            cnt_grp[g2] = lax.dot_general(
                cnt_sorted[:, g2 * G:(g2 + 1) * G], expand,
                (((1,), (0,)), ((), ())), preferred_element_type=f32,
            )

        qv = q_ref[:, :]
        lane_h = lax.broadcasted_iota(jnp.int32, (1, HD), 1) // D
        q_all = jnp.concatenate(
            [jnp.where(lane_h == h, qv, 0.0) for h in range(H)], axis=0
        )

        copy.wait()
        ng = pg_smem[1, 0]

        def issue(g, buf):
            for j in range(G):
                pg = pg_smem[0, g * G + j]
                pltpu.make_async_copy(
                    k_any.at[pg], kbuf.at[buf, j], copy_sems.at[buf]
                ).start()
                pltpu.make_async_copy(
                    v_any.at[pg], vbuf.at[buf, j], copy_sems.at[buf]
                ).start()

        def wait_group(buf):
            for j in range(G):
                pltpu.make_async_copy(
                    k_any.at[0], kbuf.at[buf, j], copy_sems.at[buf]
                ).wait()
                pltpu.make_async_copy(
                    v_any.at[0], vbuf.at[buf, j], copy_sems.at[buf]
                ).wait()

        @pl.when(ng > 0)
        def _prologue():
            issue(0, 0)

        def loop_body(g, _):
            buf = lax.rem(g, 2)
            nxt = lax.rem(g + 1, 2)

            @pl.when(g + 1 < ng)
            def _issue_next():
                issue(g + 1, nxt)

            wait_group(buf)

            kc = kbuf[buf].reshape(T, HD)
            s = lax.dot_general(
                q_all, kc, (((1,), (1,)), ((), ())),
                preferred_element_type=f32,
            ) * scale
            cnt_g = cnt_grp[g]
            cnt_all = jnp.concatenate([cnt_g] * H, axis=0)
            p = cnt_all * jnp.exp(s)
            vc = vbuf[buf].reshape(T, HD)
            o_full = lax.dot_general(
                p, vc, (((1,), (0,)), ((), ())),
                preferred_element_type=f32,
            )
            acc_ref[:, D:2 * D] += jnp.sum(p, axis=1, keepdims=True)
            for h in range(H):
                acc_ref[h * B:(h + 1) * B, 0:D] += (
                    o_full[h * B:(h + 1) * B, h * D:(h + 1) * D]
                )
            return 0

        lax.fori_loop(0, ng, loop_body, 0)

        for h in range(H):
            comm_ref[0, :, h, :] = acc_ref[h * B:(h + 1) * B, :]

        barrier = pltpu.get_barrier_semaphore()
        for t in range(1, N_DEV):
            pl.semaphore_signal(
                barrier, inc=1,
                device_id=((my + t) % N_DEV,),
                device_id_type=pl.DeviceIdType.MESH,
            )
        pl.semaphore_wait(barrier, N_DEV - 1)

        rdmas = []
        for t in range(1, N_DEV):
            r = pltpu.make_async_remote_copy(
                src_ref=comm_ref.at[0],
                dst_ref=comm_ref.at[t],
                send_sem=send_sems.at[t],
                recv_sem=recv_sems.at[t],
                device_id=((my + t) % N_DEV,),
                device_id_type=pl.DeviceIdType.MESH,
            )
            r.start()
            rdmas.append(r)
        for r in rdmas:
            r.wait()

        num = jnp.zeros((B, H, D), f32)
        den = jnp.zeros((B, H, D), f32)
        for t in range(N_DEV):
            num = num + comm_ref[t, :, :, 0:D]
            den = den + comm_ref[t, :, :, D:2 * D]
        out_ref[:, 0, :, :] = num / den

    Qf = Q.reshape(B, HD)
    Kf = K.reshape(NLOC, BS, HD)
    Vf = V.reshape(NLOC, BS, HD)
    lens2 = lens.reshape(B, 1)

    return pl.pallas_call(
        body,
        in_specs=[
            pl.BlockSpec(memory_space=pltpu.MemorySpace.VMEM),
            pl.BlockSpec(memory_space=pl.ANY),
            pl.BlockSpec(memory_space=pl.ANY),
            pl.BlockSpec(memory_space=pltpu.MemorySpace.VMEM),
            pl.BlockSpec(memory_space=pltpu.MemorySpace.VMEM),
        ],
        out_specs=pl.BlockSpec(memory_space=pltpu.MemorySpace.VMEM),
        out_shape=jax.ShapeDtypeStruct((B, 1, H, D), f32),
        scratch_shapes=[
            pltpu.VMEM((2, G, BS, HD), f32),
            pltpu.VMEM((2, G, BS, HD), f32),
            pltpu.VMEM((R, 2 * D), f32),
            pltpu.VMEM((N_DEV, B, H, 2 * D), f32),
            pltpu.VMEM((NG, B, T), f32),
            pltpu.VMEM((2, NLOC), jnp.int32),
            pltpu.SMEM((2, NLOC), jnp.int32),
            pltpu.SemaphoreType.DMA,
            pltpu.SemaphoreType.DMA((2,)),
            pltpu.SemaphoreType.DMA((N_DEV,)),
            pltpu.SemaphoreType.DMA((N_DEV,)),
        ],
        compiler_params=pltpu.CompilerParams(collective_id=0),
    )(Qf, Kf, Vf, bt, lens2)


# device time: 99799 ns/iter; 1.2154x vs baseline; 1.0179x over previous
import jax
import jax.numpy as jnp
from jax import lax
from jax.experimental import pallas as pl
from jax.experimental.pallas import tpu as pltpu

N_DEV = 4


def kernel(Q, K, V, bt, lens):
    B, _, H, D = Q.shape
    NLOC, BS, _, _ = K.shape
    NB = bt.shape[1]
    G = 64
    NG = NLOC // G
    T = G * BS
    HD = H * D
    R = H * B
    scale = D ** -0.5
    f32 = jnp.float32

    def body(q_ref, k_any, v_any, btT_ref, lens_ref, out_ref,
             kbuf, vbuf, acc_ref, comm_ref, cnt_grp, pg_vmem, pg_smem,
             stage_sem, copy_sems, send_sems, recv_sems):
        my = lax.axis_index("i")

        acc_ref[:, :] = jnp.zeros((R, 2 * D), f32)

        base = my * NLOC
        pid_col = base + lax.broadcasted_iota(jnp.int32, (NLOC, 1), 0)
        klane = lax.broadcasted_iota(jnp.int32, (1, NB), 1)
        cols = []
        for i in range(B):
            btrow = btT_ref[i:i + 1, :]
            li = lens_ref[i:i + 1, 0:1]
            eq = (btrow == pid_col) & (klane < li)
            cols.append(jnp.sum(eq.astype(f32), axis=1, keepdims=True))
        cntT = jnp.concatenate(cols, axis=1)
        ref_col = (jnp.sum(cntT, axis=1, keepdims=True) > 0.0).astype(f32)

        lt = (
            lax.broadcasted_iota(jnp.int32, (NLOC, NLOC), 1)
            < lax.broadcasted_iota(jnp.int32, (NLOC, NLOC), 0)
        ).astype(f32)
        rank_col = lax.dot_general(
            lt, ref_col, (((1,), (0,)), ((), ())),
            preferred_element_type=f32,
        )
        slot_row = lax.broadcasted_iota(jnp.int32, (1, NLOC), 1)
        sel = jnp.where(
            (rank_col == slot_row.astype(f32)) & (ref_col > 0.0), 1.0, 0.0
        )
        p_lane = lax.broadcasted_iota(jnp.int32, (1, NLOC), 1)
        p_hi = (p_lane // 256).astype(f32)
        p_lo = (p_lane % 256).astype(f32)
        pages = (
            256.0 * lax.dot_general(p_hi, sel, (((1,), (0,)), ((), ())),
                                    preferred_element_type=f32)
            + lax.dot_general(p_lo, sel, (((1,), (0,)), ((), ())),
                              preferred_element_type=f32)
        )
        n_ref = jnp.sum(ref_col)
        ng_f = jnp.floor((n_ref + (G - 1)) / G)
        pg_vmem[0:1, :] = pages.astype(jnp.int32)
        pg_vmem[1:2, :] = jnp.broadcast_to(
            ng_f.astype(jnp.int32)[None, None], (1, NLOC)
        )
        copy = pltpu.make_async_copy(pg_vmem, pg_smem, stage_sem)
        copy.start()

        def issue(g, buf):
            for j in range(G):
                pg = pg_smem[0, g * G + j]
                pltpu.make_async_copy(
                    k_any.at[pg], kbuf.at[buf, j], copy_sems.at[buf]
                ).start()
                pltpu.make_async_copy(
                    v_any.at[pg], vbuf.at[buf, j], copy_sems.at[buf]
                ).start()

        def wait_group(buf):
            for j in range(G):
                pltpu.make_async_copy(
                    k_any.at[0], kbuf.at[buf, j], copy_sems.at[buf]
                ).wait()
                pltpu.make_async_copy(
                    v_any.at[0], vbuf.at[buf, j], copy_sems.at[buf]
                ).wait()

        copy.wait()
        ng = pg_smem[1, 0]

        @pl.when(ng > 0)
        def _prologue():
            issue(0, 0)

        barrier = pltpu.get_barrier_semaphore()
        for t in range(1, N_DEV):
            pl.semaphore_signal(
                barrier, inc=1,
                device_id=((my + t) % N_DEV,),
                device_id_type=pl.DeviceIdType.MESH,
            )

        cnt_sorted = lax.dot_general(
            cntT, sel, (((0,), (0,)), ((), ())),
            preferred_element_type=f32,
        )
        expand = (
            lax.broadcasted_iota(jnp.int32, (G, T), 0)
            == lax.broadcasted_iota(jnp.int32, (G, T), 1) // BS
        ).astype(f32)
        for g2 in range(NG):
            cnt_grp[g2] = lax.dot_general(
                cnt_sorted[:, g2 * G:(g2 + 1) * G], expand,
                (((1,), (0,)), ((), ())), preferred_element_type=f32,
            )

        qv = q_ref[:, :]
        lane_h = lax.broadcasted_iota(jnp.int32, (1, HD), 1) // D
        q_all = jnp.concatenate(
            [jnp.where(lane_h == h, qv, 0.0) for h in range(H)], axis=0
        )

        def loop_body(g, _):
            buf = lax.rem(g, 2)
            nxt = lax.rem(g + 1, 2)

            @pl.when(g + 1 < ng)
            def _issue_next():
                issue(g + 1, nxt)

            wait_group(buf)

            kc = kbuf[buf].reshape(T, HD)
            s = lax.dot_general(
                q_all, kc, (((1,), (1,)), ((), ())),
                preferred_element_type=f32,
            ) * scale
            cnt_g = cnt_grp[g]
            cnt_all = jnp.concatenate([cnt_g] * H, axis=0)
            p = cnt_all * jnp.exp(s)
            vc = vbuf[buf].reshape(T, HD)
            o_full = lax.dot_general(
                p, vc, (((1,), (0,)), ((), ())),
                preferred_element_type=f32,
            )
            acc_ref[:, D:2 * D] += jnp.sum(p, axis=1, keepdims=True)
            for h in range(H):
                acc_ref[h * B:(h + 1) * B, 0:D] += (
                    o_full[h * B:(h + 1) * B, h * D:(h + 1) * D]
                )
            return 0

        lax.fori_loop(0, ng, loop_body, 0)

        for h in range(H):
            comm_ref[0, :, h, :] = acc_ref[h * B:(h + 1) * B, :]

        pl.semaphore_wait(barrier, N_DEV - 1)

        rdmas = []
        for t in range(1, N_DEV):
            r = pltpu.make_async_remote_copy(
                src_ref=comm_ref.at[0],
                dst_ref=comm_ref.at[t],
                send_sem=send_sems.at[t],
                recv_sem=recv_sems.at[t],
                device_id=((my + t) % N_DEV,),
                device_id_type=pl.DeviceIdType.MESH,
            )
            r.start()
            rdmas.append(r)
        for r in rdmas:
            r.wait()

        num = jnp.zeros((B, H, D), f32)
        den = jnp.zeros((B, H, D), f32)
        for t in range(N_DEV):
            num = num + comm_ref[t, :, :, 0:D]
            den = den + comm_ref[t, :, :, D:2 * D]
        out_ref[:, 0, :, :] = num / den

    Qf = Q.reshape(B, HD)
    Kf = K.reshape(NLOC, BS, HD)
    Vf = V.reshape(NLOC, BS, HD)
    lens2 = lens.reshape(B, 1)

    return pl.pallas_call(
        body,
        in_specs=[
            pl.BlockSpec(memory_space=pltpu.MemorySpace.VMEM),
            pl.BlockSpec(memory_space=pl.ANY),
            pl.BlockSpec(memory_space=pl.ANY),
            pl.BlockSpec(memory_space=pltpu.MemorySpace.VMEM),
            pl.BlockSpec(memory_space=pltpu.MemorySpace.VMEM),
        ],
        out_specs=pl.BlockSpec(memory_space=pltpu.MemorySpace.VMEM),
        out_shape=jax.ShapeDtypeStruct((B, 1, H, D), f32),
        scratch_shapes=[
            pltpu.VMEM((2, G, BS, HD), f32),
            pltpu.VMEM((2, G, BS, HD), f32),
            pltpu.VMEM((R, 2 * D), f32),
            pltpu.VMEM((N_DEV, B, H, 2 * D), f32),
            pltpu.VMEM((NG, B, T), f32),
            pltpu.VMEM((2, NLOC), jnp.int32),
            pltpu.SMEM((2, NLOC), jnp.int32),
            pltpu.SemaphoreType.DMA,
            pltpu.SemaphoreType.DMA((2,)),
            pltpu.SemaphoreType.DMA((N_DEV,)),
            pltpu.SemaphoreType.DMA((N_DEV,)),
        ],
        compiler_params=pltpu.CompilerParams(collective_id=0),
    )(Qf, Kf, Vf, bt, lens2)


# device time: 97013 ns/iter; 1.2503x vs baseline; 1.0287x over previous
import jax
import jax.numpy as jnp
from jax import lax
from jax.experimental import pallas as pl
from jax.experimental.pallas import tpu as pltpu

N_DEV = 4


def kernel(Q, K, V, bt, lens):
    B, _, H, D = Q.shape
    NLOC, BS, _, _ = K.shape
    NB = bt.shape[1]
    G = 64
    NG = NLOC // G
    T = G * BS
    HD = H * D
    R = H * B
    scale = D ** -0.5
    f32 = jnp.float32

    def body(q_ref, k_any, v_any, btT_ref, lens_ref, out_ref,
             kbuf, vbuf, acc_ref, comm_ref, cnt_grp, pg_vmem, pg_smem,
             stage_sem, copy_sems, send_sems, recv_sems):
        my = lax.axis_index("i")

        for j in range(G):
            pltpu.make_async_copy(
                k_any.at[j], kbuf.at[0, j], copy_sems.at[0]
            ).start()
            pltpu.make_async_copy(
                v_any.at[j], vbuf.at[0, j], copy_sems.at[0]
            ).start()

        barrier = pltpu.get_barrier_semaphore()
        for t in range(1, N_DEV):
            pl.semaphore_signal(
                barrier, inc=1,
                device_id=((my + t) % N_DEV,),
                device_id_type=pl.DeviceIdType.MESH,
            )

        acc_ref[:, :] = jnp.zeros((R, 2 * D), f32)

        base = my * NLOC
        pid_col = base + lax.broadcasted_iota(jnp.int32, (NLOC, 1), 0)
        klane = lax.broadcasted_iota(jnp.int32, (1, NB), 1)
        cols = []
        for i in range(B):
            btrow = btT_ref[i:i + 1, :]
            li = lens_ref[i:i + 1, 0:1]
            eq = (btrow == pid_col) & (klane < li)
            cols.append(jnp.sum(eq.astype(f32), axis=1, keepdims=True))
        cntT = jnp.concatenate(cols, axis=1)
        tail_page = (
            lax.broadcasted_iota(jnp.int32, (NLOC, 1), 0) >= G
        )
        ref_col = jnp.where(
            tail_page & (jnp.sum(cntT, axis=1, keepdims=True) > 0.0),
            1.0, 0.0,
        )

        lt = (
            lax.broadcasted_iota(jnp.int32, (NLOC, NLOC), 1)
            < lax.broadcasted_iota(jnp.int32, (NLOC, NLOC), 0)
        ).astype(f32)
        rank_col = lax.dot_general(
            lt, ref_col, (((1,), (0,)), ((), ())),
            preferred_element_type=f32,
        )
        slot_row = lax.broadcasted_iota(jnp.int32, (1, NLOC), 1)
        sel = jnp.where(
            (rank_col == slot_row.astype(f32)) & (ref_col > 0.0), 1.0, 0.0
        )
        p_lane = lax.broadcasted_iota(jnp.int32, (1, NLOC), 1)
        p_hi = (p_lane // 256).astype(f32)
        p_lo = (p_lane % 256).astype(f32)
        pages = (
            256.0 * lax.dot_general(p_hi, sel, (((1,), (0,)), ((), ())),
                                    preferred_element_type=f32)
            + lax.dot_general(p_lo, sel, (((1,), (0,)), ((), ())),
                              preferred_element_type=f32)
        )
        n_ref = jnp.sum(ref_col)
        ng_f = 1.0 + jnp.floor((n_ref + (G - 1)) / G)
        pg_vmem[0:1, :] = pages.astype(jnp.int32)
        pg_vmem[1:2, :] = jnp.broadcast_to(
            ng_f.astype(jnp.int32)[None, None], (1, NLOC)
        )
        copy = pltpu.make_async_copy(pg_vmem, pg_smem, stage_sem)
        copy.start()

        def issue(g, buf):
            for j in range(G):
                pg = pg_smem[0, (g - 1) * G + j]
                pltpu.make_async_copy(
                    k_any.at[pg], kbuf.at[buf, j], copy_sems.at[buf]
                ).start()
                pltpu.make_async_copy(
                    v_any.at[pg], vbuf.at[buf, j], copy_sems.at[buf]
                ).start()

        def wait_group(buf):
            for j in range(G):
                pltpu.make_async_copy(
                    k_any.at[0], kbuf.at[buf, j], copy_sems.at[buf]
                ).wait()
                pltpu.make_async_copy(
                    v_any.at[0], vbuf.at[buf, j], copy_sems.at[buf]
                ).wait()

        cnt_sorted = lax.dot_general(
            cntT, sel, (((0,), (0,)), ((), ())),
            preferred_element_type=f32,
        )
        expand = (
            lax.broadcasted_iota(jnp.int32, (G, T), 0)
            == lax.broadcasted_iota(jnp.int32, (G, T), 1) // BS
        ).astype(f32)
        cnt_grp[0] = lax.dot_general(
            cntT[0:G, :], expand, (((0,), (0,)), ((), ())),
            preferred_element_type=f32,
        )
        for g2 in range(1, NG):
            cnt_grp[g2] = lax.dot_general(
                cnt_sorted[:, (g2 - 1) * G:g2 * G], expand,
                (((1,), (0,)), ((), ())), preferred_element_type=f32,
            )

        copy.wait()
        ng = pg_smem[1, 0]

        qv = q_ref[:, :]
        lane_h = lax.broadcasted_iota(jnp.int32, (1, HD), 1) // D
        q_all = jnp.concatenate(
            [jnp.where(lane_h == h, qv, 0.0) for h in range(H)], axis=0
        )

        def loop_body(g, _):
            buf = lax.rem(g, 2)
            nxt = lax.rem(g + 1, 2)

            @pl.when(g + 1 < ng)
            def _issue_next():
                issue(g + 1, nxt)

            wait_group(buf)

            kc = kbuf[buf].reshape(T, HD)
            s = lax.dot_general(
                q_all, kc, (((1,), (1,)), ((), ())),
                preferred_element_type=f32,
            ) * scale
            cnt_g = cnt_grp[g]
            cnt_all = jnp.concatenate([cnt_g] * H, axis=0)
            p = cnt_all * jnp.exp(s)
            vc = vbuf[buf].reshape(T, HD)
            o_full = lax.dot_general(
                p, vc, (((1,), (0,)), ((), ())),
                preferred_element_type=f32,
            )
            acc_ref[:, D:2 * D] += jnp.sum(p, axis=1, keepdims=True)
            for h in range(H):
                acc_ref[h * B:(h + 1) * B, 0:D] += (
                    o_full[h * B:(h + 1) * B, h * D:(h + 1) * D]
                )
            return 0

        lax.fori_loop(0, ng, loop_body, 0)

        for h in range(H):
            comm_ref[0, :, h, :] = acc_ref[h * B:(h + 1) * B, :]

        pl.semaphore_wait(barrier, N_DEV - 1)

        rdmas = []
        for t in range(1, N_DEV):
            r = pltpu.make_async_remote_copy(
                src_ref=comm_ref.at[0],
                dst_ref=comm_ref.at[t],
                send_sem=send_sems.at[t],
                recv_sem=recv_sems.at[t],
                device_id=((my + t) % N_DEV,),
                device_id_type=pl.DeviceIdType.MESH,
            )
            r.start()
            rdmas.append(r)
        for r in rdmas:
            r.wait()

        num = jnp.zeros((B, H, D), f32)
        den = jnp.zeros((B, H, D), f32)
        for t in range(N_DEV):
            num = num + comm_ref[t, :, :, 0:D]
            den = den + comm_ref[t, :, :, D:2 * D]
        out_ref[:, 0, :, :] = num / den

    Qf = Q.reshape(B, HD)
    Kf = K.reshape(NLOC, BS, HD)
    Vf = V.reshape(NLOC, BS, HD)
    lens2 = lens.reshape(B, 1)

    return pl.pallas_call(
        body,
        in_specs=[
            pl.BlockSpec(memory_space=pltpu.MemorySpace.VMEM),
            pl.BlockSpec(memory_space=pl.ANY),
            pl.BlockSpec(memory_space=pl.ANY),
            pl.BlockSpec(memory_space=pltpu.MemorySpace.VMEM),
            pl.BlockSpec(memory_space=pltpu.MemorySpace.VMEM),
        ],
        out_specs=pl.BlockSpec(memory_space=pltpu.MemorySpace.VMEM),
        out_shape=jax.ShapeDtypeStruct((B, 1, H, D), f32),
        scratch_shapes=[
            pltpu.VMEM((2, G, BS, HD), f32),
            pltpu.VMEM((2, G, BS, HD), f32),
            pltpu.VMEM((R, 2 * D), f32),
            pltpu.VMEM((N_DEV, B, H, 2 * D), f32),
            pltpu.VMEM((NG, B, T), f32),
            pltpu.VMEM((2, NLOC), jnp.int32),
            pltpu.SMEM((2, NLOC), jnp.int32),
            pltpu.SemaphoreType.DMA,
            pltpu.SemaphoreType.DMA((2,)),
            pltpu.SemaphoreType.DMA((N_DEV,)),
            pltpu.SemaphoreType.DMA((N_DEV,)),
        ],
        compiler_params=pltpu.CompilerParams(collective_id=0),
    )(Qf, Kf, Vf, bt, lens2)
